# baseline (speedup 1.0000x reference)
import functools

import jax
import jax.numpy as jnp
import numpy as np
from jax.experimental import pallas as pl
from jax.experimental.pallas import tpu as pltpu
from jax.experimental.pallas import tpu_sc as plsc

B = 4
S = 4096
HIDDEN = 2048
N = B * S
CAPACITY = 0.7
TEMPERATURE = 0.5
LB_WEIGHT = 0.005
Z_LOSS_WEIGHT = 5e-06
K = max(1, min(int(CAPACITY * N), N))
CHUNK = 1024
NSTEP = N // CHUNK

NWORK = 16
PER = N // NWORK
NV = PER // 16

_MSB = -2147483648
HBITS = 272


def _tc_kernel(h_ref, w_ref, u_ref, b_ref, key_ref, aux_ref,
               asig_ref, asq_ref):
    i = pl.program_id(0)
    h = h_ref[0].astype(jnp.bfloat16)
    w8 = jnp.broadcast_to(w_ref[...], (8, HIDDEN)).astype(jnp.bfloat16)
    o = jax.lax.dot_general(
        w8, h,
        (((1,), (1,)), ((), ())),
        preferred_element_type=jnp.float32,
    )
    s8 = o[0:1, :].reshape(8, 128) + b_ref[0]

    u = u_ref[...]
    gumbel = -jnp.log(-jnp.log(u + 1e-10) + 1e-10)
    noisy = (s8 + gumbel) / TEMPERATURE
    bits = jax.lax.bitcast_convert_type(noisy, jnp.int32)
    ukey8 = jnp.where(bits < 0, ~bits, bits ^ _MSB)
    key_ref[pl.ds(i * 8, 8), :] = ukey8

    sig8 = jax.nn.sigmoid(s8)
    sq8 = s8 * s8

    @pl.when(i == 0)
    def _():
        asig_ref[...] = sig8
        asq_ref[...] = sq8

    @pl.when(i > 0)
    def _():
        asig_ref[...] += sig8
        asq_ref[...] += sq8

    @pl.when(i == NSTEP - 1)
    def _():
        p = jnp.sum(asig_ref[...]) / N
        z = jnp.sum(asq_ref[...]) / N
        f = np.float32(np.float32(K) / np.float32(N))
        lb = (f - CAPACITY) ** 2 + (p - CAPACITY) ** 2
        aux_ref[...] = (LB_WEIGHT * lb + Z_LOSS_WEIGHT * z).reshape(1, 1)


def _srl(x, n):
    return jax.lax.shift_right_logical(x, jnp.int32(n) if isinstance(n, int) else n)


def _sc_topk_body(keys_hbm, out_hbm, keys_v, lhist, mask_v, idx2d, ones_v,
                  zeros_v, big_v, h0, h1, h2, h3, h4, h5):
    wid = jax.lax.axis_index("s")
    base = wid * PER
    hists = (h0, h1, h2, h3, h4, h5)
    pltpu.sync_copy(keys_hbm.at[pl.ds(base, PER)], keys_v)

    iota16 = jax.lax.iota(jnp.int32, 16)

    def cbody(j, _):
        ones_v[pl.ds(j * 16, 16)] = jnp.ones((16,), jnp.float32)
        return 0
    jax.lax.fori_loop(0, 8, cbody, 0)

    def zbody(j, _):
        zeros_v[pl.ds(j * 16, 16)] = jnp.zeros((16,), jnp.float32)
        return 0
    jax.lax.fori_loop(0, HBITS // 16, zbody, 0)

    for hh in hists:
        pltpu.sync_copy(zeros_v, hh.at[pl.ds(wid * HBITS, HBITS)])
    plsc.subcore_barrier()

    def _vsum16(x):
        dnums = jax.lax.GatherDimensionNumbers(
            offset_dims=(), collapsed_slice_dims=(0,), start_index_map=(0,))
        for k in (8, 4, 2, 1):
            perm = (iota16 ^ k).reshape(16, 1)
            x = x + jax.lax.gather(
                x, perm, dnums, (1,),
                mode=jax.lax.GatherScatterMode.PROMISE_IN_BOUNDS)
        return x[0]

    def suffix_count(b):
        def sbody(j, acc):
            biota = j * 16 + iota16
            hj = lhist[pl.ds(j * 16, 16)]
            return acc + jnp.where(biota >= b, hj, np.float32(0.0))
        acc = jax.lax.fori_loop(0, 16, sbody, jnp.zeros((16,), jnp.float32))
        return _vsum16(acc)

    def find_digit(kp, nbits):
        kpf = kp.astype(jnp.float32)

        def fbody(step, bcur):
            bit = jnp.left_shift(jnp.int32(1), nbits - 1 - step)
            cand = bcur | bit
            return jnp.where(suffix_count(cand) >= kpf, cand, bcur)

        bstar = jax.lax.fori_loop(0, nbits, fbody, jnp.int32(0))
        return bstar, kp - suffix_count(bstar + 1).astype(jnp.int32)

    def run_round(r, bucket_of_vreg):
        def bbody(v, _):
            byte = bucket_of_vreg(v) + wid * HBITS
            idx2d[v // 8, pl.ds((v % 8) * 16, 16)] = byte
            return 0
        jax.lax.fori_loop(0, NV, bbody, 0)
        for j in range(8):
            pltpu.sync_copy(ones_v, hists[r].at[idx2d.at[j]], add=True)
        plsc.subcore_barrier()
        pltpu.sync_copy(hists[r], big_v)

        def zb(j, _):
            lhist[pl.ds(j * 16, 16)] = jnp.zeros((16,), jnp.float32)
            return 0
        jax.lax.fori_loop(0, HBITS // 16, zb, 0)

        def mb(w, _):
            def mj(j, _):
                sl = pl.ds(j * 16, 16)
                lhist[sl] = lhist[sl] + big_v[pl.ds(w * HBITS + j * 16, 16)]
                return 0
            jax.lax.fori_loop(0, HBITS // 16, mj, 0)
            return 0
        jax.lax.fori_loop(0, NWORK, mb, 0)

    kp = jnp.int32(K)
    tu = jnp.int32(0)

    for r in range(4):
        shift = 24 - 8 * r

        def bucket(v, _r=r, _shift=shift, _tu=tu):
            uk = keys_v[pl.ds(v * 16, 16)]
            byte = _srl(uk, _shift) & 255
            if _r == 0:
                return byte
            matched = _srl(uk, _shift + 8) == _srl(_tu, _shift + 8)
            return jnp.where(matched, byte, jnp.int32(256))

        run_round(r, bucket)
        bstar, kp = find_digit(kp, 8)
        tu = tu | jnp.left_shift(bstar, shift)

    b4 = jnp.int32(0)
    for r in range(4, 6):

        def bucket(v, _r=r, _tu=tu, _b4=b4):
            uk = keys_v[pl.ds(v * 16, 16)]
            rq = (N - 1) - (base + v * 16 + iota16)
            matched = uk == _tu
            if _r == 4:
                byte = _srl(rq, 7)
            else:
                matched = matched & (_srl(rq, 7) == _b4)
                byte = rq & 127
            return jnp.where(matched, byte, jnp.int32(256))

        run_round(r, bucket)
        bstar, kp = find_digit(kp, 7)
        if r == 4:
            b4 = bstar
        else:
            rqstar = jnp.left_shift(b4, 7) | bstar

    st = tu ^ jnp.int32(_MSB)

    def wbody(v, _):
        uk = keys_v[pl.ds(v * 16, 16)]
        sx = uk ^ jnp.int32(_MSB)
        rq = (N - 1) - (base + v * 16 + iota16)
        sel = (sx > st) | ((sx == st) & (rq >= rqstar))
        mask_v[pl.ds(v * 16, 16)] = jnp.where(sel, 1, 0).astype(jnp.int32)
        return 0
    jax.lax.fori_loop(0, NV, wbody, 0)

    pltpu.sync_copy(mask_v, out_hbm.at[pl.ds(base, PER)])


_sc_topk = functools.partial(
    pl.kernel,
    out_type=jax.ShapeDtypeStruct((N,), jnp.int32),
    mesh=plsc.VectorSubcoreMesh(
        core_axis_name="c", subcore_axis_name="s", num_cores=1),
    scratch_types=[
        pltpu.VMEM((PER,), jnp.int32),
        pltpu.VMEM((HBITS,), jnp.float32),
        pltpu.VMEM((PER,), jnp.int32),
        pltpu.VMEM((8, 128), jnp.int32),
        pltpu.VMEM((128,), jnp.float32),
        pltpu.VMEM((HBITS,), jnp.float32),
        pltpu.VMEM((NWORK * HBITS,), jnp.float32),
    ] + [pltpu.VMEM_SHARED((NWORK * HBITS,), jnp.float32) for _ in range(6)],
)(_sc_topk_body)


@jax.jit
def kernel(hidden_states, active_mask, router_w, router_b, gumbel_u):
    del active_mask
    nper = S // CHUNK
    ukey128, aux = pl.pallas_call(
        _tc_kernel,
        grid=(NSTEP,),
        in_specs=[
            pl.BlockSpec((1, CHUNK, HIDDEN), lambda i: (i // nper, i % nper, 0)),
            pl.BlockSpec((1, HIDDEN), lambda i: (0, 0)),
            pl.BlockSpec((8, 128), lambda i: (i, 0)),
            pl.BlockSpec(memory_space=pltpu.SMEM),
        ],
        out_specs=(
            pl.BlockSpec((128, 128), lambda i: (0, 0)),
            pl.BlockSpec((1, 1), lambda i: (0, 0)),
        ),
        out_shape=(
            jax.ShapeDtypeStruct((128, 128), jnp.int32),
            jax.ShapeDtypeStruct((1, 1), jnp.float32),
        ),
        scratch_shapes=[
            pltpu.VMEM((8, 128), jnp.float32),
            pltpu.VMEM((8, 128), jnp.float32),
        ],
    )(hidden_states, router_w, gumbel_u.reshape(128, 128), router_b)

    mask_i32 = _sc_topk(ukey128.reshape(N))
    ffn_mask = (mask_i32 != 0).reshape(B, S)
    return ffn_mask, aux[0, 0]


# speedup vs baseline: 1.0036x; 1.0036x over previous
import functools

import jax
import jax.numpy as jnp
import numpy as np
from jax.experimental import pallas as pl
from jax.experimental.pallas import tpu as pltpu
from jax.experimental.pallas import tpu_sc as plsc

B = 4
S = 4096
HIDDEN = 2048
N = B * S
CAPACITY = 0.7
TEMPERATURE = 0.5
LB_WEIGHT = 0.005
Z_LOSS_WEIGHT = 5e-06
K = max(1, min(int(CAPACITY * N), N))
CHUNK = 1024
NSTEP = N // CHUNK

NWORK = 16
PER = N // NWORK
NV = PER // 16

_MSB = -2147483648
HBITS = 272


def _tc_kernel(h_ref, w_ref, u_ref, b_ref, key_ref, aux_ref,
               asig_ref, asq_ref):
    i = pl.program_id(0)
    h = h_ref[0].astype(jnp.bfloat16)
    w8 = jnp.broadcast_to(w_ref[...], (8, HIDDEN)).astype(jnp.bfloat16)
    o = jax.lax.dot_general(
        w8, h,
        (((1,), (1,)), ((), ())),
        preferred_element_type=jnp.float32,
    )
    s8 = o[0:1, :].reshape(8, 128) + b_ref[0]

    u = u_ref[...]
    gumbel = -jnp.log(-jnp.log(u + 1e-10) + 1e-10)
    noisy = (s8 + gumbel) / TEMPERATURE
    bits = jax.lax.bitcast_convert_type(noisy, jnp.int32)
    ukey8 = jnp.where(bits < 0, ~bits, bits ^ _MSB)
    key_ref[pl.ds(i * 8, 8), :] = ukey8

    sig8 = jax.nn.sigmoid(s8)
    sq8 = s8 * s8

    @pl.when(i == 0)
    def _():
        asig_ref[...] = sig8
        asq_ref[...] = sq8

    @pl.when(i > 0)
    def _():
        asig_ref[...] += sig8
        asq_ref[...] += sq8

    @pl.when(i == NSTEP - 1)
    def _():
        p = jnp.sum(asig_ref[...]) / N
        z = jnp.sum(asq_ref[...]) / N
        f = np.float32(np.float32(K) / np.float32(N))
        lb = (f - CAPACITY) ** 2 + (p - CAPACITY) ** 2
        aux_ref[...] = (LB_WEIGHT * lb + Z_LOSS_WEIGHT * z).reshape(1, 1)


def _srl(x, n):
    return jax.lax.shift_right_logical(x, jnp.int32(n) if isinstance(n, int) else n)


def _sc_topk_body(keys_hbm, out_hbm, keys_v, lhist, mask_v, idx2d, ones_v,
                  zeros_v, big_v, h0, h1, h2, h3, h4, h5, sem):
    wid = jax.lax.axis_index("s")
    base = wid * PER
    hists = (h0, h1, h2, h3, h4, h5)
    pltpu.sync_copy(keys_hbm.at[pl.ds(base, PER)], keys_v)

    iota16 = jax.lax.iota(jnp.int32, 16)

    def cbody(j, _):
        ones_v[pl.ds(j * 16, 16)] = jnp.ones((16,), jnp.float32)
        return 0
    jax.lax.fori_loop(0, 8, cbody, 0)

    def zbody(j, _):
        zeros_v[pl.ds(j * 16, 16)] = jnp.zeros((16,), jnp.float32)
        return 0
    jax.lax.fori_loop(0, HBITS // 16, zbody, 0)

    for hh in hists:
        pltpu.sync_copy(zeros_v, hh.at[pl.ds(wid * HBITS, HBITS)])
    plsc.subcore_barrier()

    def _vsum16(x):
        dnums = jax.lax.GatherDimensionNumbers(
            offset_dims=(), collapsed_slice_dims=(0,), start_index_map=(0,))
        for k in (8, 4, 2, 1):
            perm = (iota16 ^ k).reshape(16, 1)
            x = x + jax.lax.gather(
                x, perm, dnums, (1,),
                mode=jax.lax.GatherScatterMode.PROMISE_IN_BOUNDS)
        return x[0]

    def suffix_count(b):
        def sbody(j, acc):
            biota = j * 16 + iota16
            hj = lhist[pl.ds(j * 16, 16)]
            return acc + jnp.where(biota >= b, hj, np.float32(0.0))
        acc = jax.lax.fori_loop(0, 16, sbody, jnp.zeros((16,), jnp.float32))
        return _vsum16(acc)

    def find_digit(kp, nbits):
        kpf = kp.astype(jnp.float32)

        def fbody(step, bcur):
            bit = jnp.left_shift(jnp.int32(1), nbits - 1 - step)
            cand = bcur | bit
            return jnp.where(suffix_count(cand) >= kpf, cand, bcur)

        bstar = jax.lax.fori_loop(0, nbits, fbody, jnp.int32(0))
        return bstar, kp - suffix_count(bstar + 1).astype(jnp.int32)

    def run_round(r, bucket_of_vreg):
        def bbody(v, _):
            byte = bucket_of_vreg(v) + wid * HBITS
            idx2d[v // 8, pl.ds((v % 8) * 16, 16)] = byte
            return 0
        jax.lax.fori_loop(0, NV, bbody, 0)
        handles = [
            pltpu.async_copy(ones_v, hists[r].at[idx2d.at[j]], sem, add=True)
            for j in range(8)
        ]
        for hnd in handles:
            hnd.wait()
        plsc.subcore_barrier()
        pltpu.sync_copy(hists[r], big_v)

        def zb(j, _):
            lhist[pl.ds(j * 16, 16)] = jnp.zeros((16,), jnp.float32)
            return 0
        jax.lax.fori_loop(0, HBITS // 16, zb, 0)

        def mb(w, _):
            def mj(j, _):
                sl = pl.ds(j * 16, 16)
                lhist[sl] = lhist[sl] + big_v[pl.ds(w * HBITS + j * 16, 16)]
                return 0
            jax.lax.fori_loop(0, HBITS // 16, mj, 0)
            return 0
        jax.lax.fori_loop(0, NWORK, mb, 0)

    kp = jnp.int32(K)
    tu = jnp.int32(0)

    for r in range(4):
        shift = 24 - 8 * r

        def bucket(v, _r=r, _shift=shift, _tu=tu):
            uk = keys_v[pl.ds(v * 16, 16)]
            byte = _srl(uk, _shift) & 255
            if _r == 0:
                return byte
            matched = _srl(uk, _shift + 8) == _srl(_tu, _shift + 8)
            return jnp.where(matched, byte, jnp.int32(256))

        run_round(r, bucket)
        bstar, kp = find_digit(kp, 8)
        tu = tu | jnp.left_shift(bstar, shift)

    b4 = jnp.int32(0)
    for r in range(4, 6):

        def bucket(v, _r=r, _tu=tu, _b4=b4):
            uk = keys_v[pl.ds(v * 16, 16)]
            rq = (N - 1) - (base + v * 16 + iota16)
            matched = uk == _tu
            if _r == 4:
                byte = _srl(rq, 7)
            else:
                matched = matched & (_srl(rq, 7) == _b4)
                byte = rq & 127
            return jnp.where(matched, byte, jnp.int32(256))

        run_round(r, bucket)
        bstar, kp = find_digit(kp, 7)
        if r == 4:
            b4 = bstar
        else:
            rqstar = jnp.left_shift(b4, 7) | bstar

    st = tu ^ jnp.int32(_MSB)

    def wbody(v, _):
        uk = keys_v[pl.ds(v * 16, 16)]
        sx = uk ^ jnp.int32(_MSB)
        rq = (N - 1) - (base + v * 16 + iota16)
        sel = (sx > st) | ((sx == st) & (rq >= rqstar))
        mask_v[pl.ds(v * 16, 16)] = jnp.where(sel, 1, 0).astype(jnp.int32)
        return 0
    jax.lax.fori_loop(0, NV, wbody, 0)

    pltpu.sync_copy(mask_v, out_hbm.at[pl.ds(base, PER)])


_sc_topk = functools.partial(
    pl.kernel,
    out_type=jax.ShapeDtypeStruct((N,), jnp.int32),
    mesh=plsc.VectorSubcoreMesh(
        core_axis_name="c", subcore_axis_name="s", num_cores=1),
    scratch_types=[
        pltpu.VMEM((PER,), jnp.int32),
        pltpu.VMEM((HBITS,), jnp.float32),
        pltpu.VMEM((PER,), jnp.int32),
        pltpu.VMEM((8, 128), jnp.int32),
        pltpu.VMEM((128,), jnp.float32),
        pltpu.VMEM((HBITS,), jnp.float32),
        pltpu.VMEM((NWORK * HBITS,), jnp.float32),
    ] + [pltpu.VMEM_SHARED((NWORK * HBITS,), jnp.float32) for _ in range(6)]
      + [pltpu.SemaphoreType.DMA],
)(_sc_topk_body)


@jax.jit
def kernel(hidden_states, active_mask, router_w, router_b, gumbel_u):
    del active_mask
    nper = S // CHUNK
    ukey128, aux = pl.pallas_call(
        _tc_kernel,
        grid=(NSTEP,),
        in_specs=[
            pl.BlockSpec((1, CHUNK, HIDDEN), lambda i: (i // nper, i % nper, 0)),
            pl.BlockSpec((1, HIDDEN), lambda i: (0, 0)),
            pl.BlockSpec((8, 128), lambda i: (i, 0)),
            pl.BlockSpec(memory_space=pltpu.SMEM),
        ],
        out_specs=(
            pl.BlockSpec((128, 128), lambda i: (0, 0)),
            pl.BlockSpec((1, 1), lambda i: (0, 0)),
        ),
        out_shape=(
            jax.ShapeDtypeStruct((128, 128), jnp.int32),
            jax.ShapeDtypeStruct((1, 1), jnp.float32),
        ),
        scratch_shapes=[
            pltpu.VMEM((8, 128), jnp.float32),
            pltpu.VMEM((8, 128), jnp.float32),
        ],
    )(hidden_states, router_w, gumbel_u.reshape(128, 128), router_b)

    mask_i32 = _sc_topk(ukey128.reshape(N))
    ffn_mask = (mask_i32 != 0).reshape(B, S)
    return ffn_mask, aux[0, 0]


# speedup vs baseline: 1.0926x; 1.0887x over previous
import functools

import jax
import jax.numpy as jnp
import numpy as np
from jax.experimental import pallas as pl
from jax.experimental.pallas import tpu as pltpu
from jax.experimental.pallas import tpu_sc as plsc

B = 4
S = 4096
HIDDEN = 2048
N = B * S
CAPACITY = 0.7
TEMPERATURE = 0.5
LB_WEIGHT = 0.005
Z_LOSS_WEIGHT = 5e-06
K = max(1, min(int(CAPACITY * N), N))
CHUNK = 1024
NSTEP = N // CHUNK

NWORK = 16
PER = N // NWORK
NV = PER // 16

_MSB = -2147483648
HBITS = 272


def _tc_kernel(h_ref, w_ref, u_ref, b_ref, key_ref, aux_ref,
               asig_ref, asq_ref):
    i = pl.program_id(0)
    h = h_ref[0].astype(jnp.bfloat16)
    w8 = jnp.broadcast_to(w_ref[...], (8, HIDDEN)).astype(jnp.bfloat16)
    o = jax.lax.dot_general(
        w8, h,
        (((1,), (1,)), ((), ())),
        preferred_element_type=jnp.float32,
    )
    s8 = o[0:1, :].reshape(8, 128) + b_ref[0]

    u = u_ref[...]
    gumbel = -jnp.log(-jnp.log(u + 1e-10) + 1e-10)
    noisy = (s8 + gumbel) / TEMPERATURE
    bits = jax.lax.bitcast_convert_type(noisy, jnp.int32)
    ukey8 = jnp.where(bits < 0, ~bits, bits ^ _MSB)
    key_ref[pl.ds(i * 8, 8), :] = ukey8

    sig8 = jax.nn.sigmoid(s8)
    sq8 = s8 * s8

    @pl.when(i == 0)
    def _():
        asig_ref[...] = sig8
        asq_ref[...] = sq8

    @pl.when(i > 0)
    def _():
        asig_ref[...] += sig8
        asq_ref[...] += sq8

    @pl.when(i == NSTEP - 1)
    def _():
        p = jnp.sum(asig_ref[...]) / N
        z = jnp.sum(asq_ref[...]) / N
        f = np.float32(np.float32(K) / np.float32(N))
        lb = (f - CAPACITY) ** 2 + (p - CAPACITY) ** 2
        aux_ref[...] = (LB_WEIGHT * lb + Z_LOSS_WEIGHT * z).reshape(1, 1)


def _srl(x, n):
    return jax.lax.shift_right_logical(x, jnp.int32(n) if isinstance(n, int) else n)


def _sc_topk_body(keys_hbm, out_hbm, keys_v, lhist, mask_v, idx2d, ones_v,
                  zeros_v, big_v, h0, h1, h2, h3, h4, h5, sem):
    wid = jax.lax.axis_index("s")
    base = wid * PER
    hists = (h0, h1, h2, h3, h4, h5)
    pltpu.sync_copy(keys_hbm.at[pl.ds(base, PER)], keys_v)

    iota16 = jax.lax.iota(jnp.int32, 16)

    def cbody(j, _):
        ones_v[pl.ds(j * 16, 16)] = jnp.ones((16,), jnp.float32)
        return 0
    jax.lax.fori_loop(0, 8, cbody, 0)

    def zbody(j, _):
        zeros_v[pl.ds(j * 16, 16)] = jnp.zeros((16,), jnp.float32)
        return 0
    jax.lax.fori_loop(0, HBITS // 16, zbody, 0)

    for hh in hists:
        pltpu.sync_copy(zeros_v, hh.at[pl.ds(wid * HBITS, HBITS)])
    plsc.subcore_barrier()

    def _vsum16(x):
        dnums = jax.lax.GatherDimensionNumbers(
            offset_dims=(), collapsed_slice_dims=(0,), start_index_map=(0,))
        for k in (8, 4, 2, 1):
            perm = (iota16 ^ k).reshape(16, 1)
            x = x + jax.lax.gather(
                x, perm, dnums, (1,),
                mode=jax.lax.GatherScatterMode.PROMISE_IN_BOUNDS)
        return x[0]

    def suffix_count(b):
        def sbody(j, acc):
            biota = j * 16 + iota16
            hj = lhist[pl.ds(j * 16, 16)]
            return acc + jnp.where(biota >= b, hj, np.float32(0.0))
        acc = jax.lax.fori_loop(0, 16, sbody, jnp.zeros((16,), jnp.float32))
        return _vsum16(acc)

    def find_digit(kp, nbits):
        kpf = kp.astype(jnp.float32)

        def fbody(step, bcur):
            bit = jnp.left_shift(jnp.int32(1), nbits - 1 - step)
            cand = bcur | bit
            return jnp.where(suffix_count(cand) >= kpf, cand, bcur)

        bstar = jax.lax.fori_loop(0, nbits, fbody, jnp.int32(0))
        return bstar, kp - suffix_count(bstar + 1).astype(jnp.int32)

    def run_round(r, bucket_of_vreg):
        def bbody(v, _):
            byte = bucket_of_vreg(v) + wid * HBITS
            idx2d[v // 8, pl.ds((v % 8) * 16, 16)] = byte
            return 0
        jax.lax.fori_loop(0, NV, bbody, 0)
        handles = [
            pltpu.async_copy(ones_v, hists[r].at[idx2d.at[j]], sem, add=True)
            for j in range(8)
        ]
        for hnd in handles:
            hnd.wait()
        plsc.subcore_barrier()
        pltpu.sync_copy(hists[r], big_v)

        def zb(j, _):
            lhist[pl.ds(j * 16, 16)] = jnp.zeros((16,), jnp.float32)
            return 0
        jax.lax.fori_loop(0, HBITS // 16, zb, 0)

        def mb(w, _):
            def mj(j, _):
                sl = pl.ds(j * 16, 16)
                lhist[sl] = lhist[sl] + big_v[pl.ds(w * HBITS + j * 16, 16)]
                return 0
            jax.lax.fori_loop(0, HBITS // 16, mj, 0)
            return 0
        jax.lax.fori_loop(0, NWORK, mb, 0)

    kp = jnp.int32(K)
    tu = jnp.int32(0)

    for r in range(4):
        shift = 24 - 8 * r

        def bucket(v, _r=r, _shift=shift, _tu=tu):
            uk = keys_v[pl.ds(v * 16, 16)]
            byte = _srl(uk, _shift) & 255
            if _r == 0:
                return byte
            matched = _srl(uk, _shift + 8) == _srl(_tu, _shift + 8)
            return jnp.where(matched, byte, jnp.int32(256))

        run_round(r, bucket)
        bstar, kp = find_digit(kp, 8)
        tu = tu | jnp.left_shift(bstar, shift)

    ceq = suffix_count(bstar) - suffix_count(bstar + 1)
    do_ties = ceq > kp.astype(jnp.float32)

    @pl.when(do_ties)
    def _():
        kp2 = kp
        b4 = jnp.int32(0)
        rqs = jnp.int32(0)
        for r in range(4, 6):

            def bucket(v, _r=r, _tu=tu, _b4=b4):
                uk = keys_v[pl.ds(v * 16, 16)]
                rq = (N - 1) - (base + v * 16 + iota16)
                matched = uk == _tu
                if _r == 4:
                    byte = _srl(rq, 7)
                else:
                    matched = matched & (_srl(rq, 7) == _b4)
                    byte = rq & 127
                return jnp.where(matched, byte, jnp.int32(256))

            run_round(r, bucket)
            bst2, kp2 = find_digit(kp2, 7)
            if r == 4:
                b4 = bst2
            else:
                rqs = jnp.left_shift(b4, 7) | bst2
        idx2d[0, pl.ds(0, 16)] = jnp.broadcast_to(rqs, (16,))

    @pl.when(jnp.logical_not(do_ties))
    def _():
        idx2d[0, pl.ds(0, 16)] = jnp.zeros((16,), jnp.int32)

    rqstar = idx2d[0, pl.ds(0, 16)][0]

    st = tu ^ jnp.int32(_MSB)

    def wbody(v, _):
        uk = keys_v[pl.ds(v * 16, 16)]
        sx = uk ^ jnp.int32(_MSB)
        rq = (N - 1) - (base + v * 16 + iota16)
        sel = (sx > st) | ((sx == st) & (rq >= rqstar))
        mask_v[pl.ds(v * 16, 16)] = jnp.where(sel, 1, 0).astype(jnp.int32)
        return 0
    jax.lax.fori_loop(0, NV, wbody, 0)

    pltpu.sync_copy(mask_v, out_hbm.at[pl.ds(base, PER)])


_sc_topk = functools.partial(
    pl.kernel,
    out_type=jax.ShapeDtypeStruct((N,), jnp.int32),
    mesh=plsc.VectorSubcoreMesh(
        core_axis_name="c", subcore_axis_name="s", num_cores=1),
    scratch_types=[
        pltpu.VMEM((PER,), jnp.int32),
        pltpu.VMEM((HBITS,), jnp.float32),
        pltpu.VMEM((PER,), jnp.int32),
        pltpu.VMEM((8, 128), jnp.int32),
        pltpu.VMEM((128,), jnp.float32),
        pltpu.VMEM((HBITS,), jnp.float32),
        pltpu.VMEM((NWORK * HBITS,), jnp.float32),
    ] + [pltpu.VMEM_SHARED((NWORK * HBITS,), jnp.float32) for _ in range(6)]
      + [pltpu.SemaphoreType.DMA],
)(_sc_topk_body)


@jax.jit
def kernel(hidden_states, active_mask, router_w, router_b, gumbel_u):
    del active_mask
    nper = S // CHUNK
    ukey128, aux = pl.pallas_call(
        _tc_kernel,
        grid=(NSTEP,),
        in_specs=[
            pl.BlockSpec((1, CHUNK, HIDDEN), lambda i: (i // nper, i % nper, 0)),
            pl.BlockSpec((1, HIDDEN), lambda i: (0, 0)),
            pl.BlockSpec((8, 128), lambda i: (i, 0)),
            pl.BlockSpec(memory_space=pltpu.SMEM),
        ],
        out_specs=(
            pl.BlockSpec((128, 128), lambda i: (0, 0)),
            pl.BlockSpec((1, 1), lambda i: (0, 0)),
        ),
        out_shape=(
            jax.ShapeDtypeStruct((128, 128), jnp.int32),
            jax.ShapeDtypeStruct((1, 1), jnp.float32),
        ),
        scratch_shapes=[
            pltpu.VMEM((8, 128), jnp.float32),
            pltpu.VMEM((8, 128), jnp.float32),
        ],
    )(hidden_states, router_w, gumbel_u.reshape(128, 128), router_b)

    mask_i32 = _sc_topk(ukey128.reshape(N))
    ffn_mask = (mask_i32 != 0).reshape(B, S)
    return ffn_mask, aux[0, 0]
